# baseline (speedup 1.0000x reference)
import jax
import jax.numpy as jnp
from jax import lax
from jax.experimental import pallas as pl
from jax.experimental.pallas import tpu as pltpu, tpu_sc as plsc

_NC, _NS = 2, 16
_NW = _NC * _NS
_B, _L, _D = 8, 2048, 512
_WPB = _NW // _B
_QROWS = _L // _WPB
_NBLK = 16
_BLK = _QROWS // _NBLK
_CHUNKS = _L // 16


def _sc_body(x_hbm, at_hbm, out_hbm, at_v, perm_v, ring,
             gsem0, gsem1, gsem2, gsem3, gsem4, gsem5,
             wsem0, wsem1, wsem2, wsem3):
    cid = lax.axis_index("c")
    sid = lax.axis_index("s")
    wid = sid * _NC + cid
    b = wid // _WPB
    q = wid % _WPB

    pltpu.sync_copy(at_hbm.at[b], at_v)

    lanes = jnp.arange(16, dtype=jnp.int32)
    row_base = b * _L
    ones = jnp.full((16,), 1, jnp.int32)
    zeros = jnp.full((16,), 0, jnp.int32)
    twos = jnp.full((16,), 2, jnp.int32)

    def count_body(k, nb):
        v = at_v[pl.ds(k * 16, 16)]
        m = (v == zeros) | (v == ones) | (v == twos)
        mi = jnp.where(m, ones, zeros)
        return nb + jnp.sum(mi)

    nb = lax.fori_loop(0, _CHUNKS, count_body, jnp.int32(0))

    def perm_body(k, carry):
        bbc, nbc = carry
        v = at_v[pl.ds(k * 16, 16)]
        m = (v == zeros) | (v == ones) | (v == twos)
        mi = jnp.where(m, ones, zeros)
        cs = jnp.cumsum(mi)
        csn = lanes + ones - cs
        bb_dest = jnp.full((16,), bbc - 1, jnp.int32) + cs
        nbb_dest = jnp.full((16,), nb + nbc - 1, jnp.int32) + csn
        dest = jnp.where(m, bb_dest, nbb_dest)
        src = row_base + k * 16 + lanes
        plsc.store_scatter(perm_v, [dest], src)
        pc = jnp.sum(mi)
        return (bbc + pc, nbc + (16 - pc))

    lax.fori_loop(0, _CHUNKS, perm_body, (jnp.int32(0), jnp.int32(0)))

    out_base = row_base + q * _QROWS
    idx_base = q * _QROWS
    gsems = (gsem0, gsem1, gsem2, gsem3, gsem4, gsem5)
    wsems = (wsem0, wsem1, wsem2, wsem3)
    _NP = _NBLK // 2

    def gather(blk):
        s = blk % 6
        return pltpu.async_copy(
            x_hbm.at[perm_v.at[pl.ds(idx_base + blk * _BLK, _BLK)]],
            ring.at[pl.ds(s * _BLK, _BLK)], gsems[s])

    def write(p):
        s = (2 * p) % 6
        return pltpu.async_copy(
            ring.at[pl.ds(s * _BLK, 2 * _BLK)],
            out_hbm.at[pl.ds(out_base + p * 2 * _BLK, 2 * _BLK)],
            wsems[p % 4])

    gd = [None] * _NBLK
    wd = [None] * _NP
    gd[0] = gather(0)
    gd[1] = gather(1)
    gd[2] = gather(2)
    gd[3] = gather(3)
    for p in range(_NP):
        if p >= 1:
            wd[p - 1].wait()
        if 2 * p + 4 < _NBLK:
            gd[2 * p + 4] = gather(2 * p + 4)
        if 2 * p + 5 < _NBLK:
            gd[2 * p + 5] = gather(2 * p + 5)
        gd[2 * p].wait()
        gd[2 * p + 1].wait()
        wd[p] = write(p)
    wd[_NP - 1].wait()

def _sc_permute(x2, at32):
    mesh = plsc.VectorSubcoreMesh(core_axis_name="c", subcore_axis_name="s")
    k = pl.kernel(
        _sc_body,
        out_type=jax.ShapeDtypeStruct((_B * _L, _D), jnp.float32),
        mesh=mesh,
        compiler_params=pltpu.CompilerParams(needs_layout_passes=False),
        scratch_types=[
            pltpu.VMEM((_L,), jnp.int32),
            pltpu.VMEM((_L,), jnp.int32),
            pltpu.VMEM((6 * _BLK, _D), jnp.float32),
            pltpu.SemaphoreType.DMA,
            pltpu.SemaphoreType.DMA,
            pltpu.SemaphoreType.DMA,
            pltpu.SemaphoreType.DMA,
            pltpu.SemaphoreType.DMA,
            pltpu.SemaphoreType.DMA,
            pltpu.SemaphoreType.DMA,
            pltpu.SemaphoreType.DMA,
            pltpu.SemaphoreType.DMA,
            pltpu.SemaphoreType.DMA,

        ],
    )
    return k(x2, at32)


@jax.jit
def kernel(x, atom_type, aa_type):
    x2 = x.reshape(_B * _L, _D)
    at32 = atom_type.astype(jnp.int32)
    out = _sc_permute(x2, at32)
    return out.reshape(_B, _L, _D)


# speedup vs baseline: 1.0173x; 1.0173x over previous
import jax
import jax.numpy as jnp
from jax import lax
from jax.experimental import pallas as pl
from jax.experimental.pallas import tpu as pltpu, tpu_sc as plsc

_NC, _NS = 2, 16
_NW = _NC * _NS
_B, _L, _D = 8, 2048, 512
_WPB = _NW // _B
_QROWS = _L // _WPB
_CHUNKS = _L // 16
_SLOT = 32
_NSLOT = 6
_NRD = _QROWS // _SLOT
_SCB = 16
_SPS = _SLOT // _SCB


def _sc_body(x_hbm, at_hbm, out_hbm, at_v, dest_v, ring,
             rsem0, rsem1, rsem2, rsem3, rsem4, rsem5,
             ssem0, ssem1, ssem2, ssem3, ssem4, ssem5):
    cid = lax.axis_index("c")
    sid = lax.axis_index("s")
    wid = sid * _NC + cid
    b = wid // _WPB
    q = wid % _WPB

    pltpu.sync_copy(at_hbm.at[b], at_v)

    lanes = jnp.arange(16, dtype=jnp.int32)
    row_base = b * _L
    ones = jnp.full((16,), 1, jnp.int32)
    zeros = jnp.full((16,), 0, jnp.int32)
    twos = jnp.full((16,), 2, jnp.int32)

    def count_body(k, nb):
        v = at_v[pl.ds(k * 16, 16)]
        m = (v == zeros) | (v == ones) | (v == twos)
        mi = jnp.where(m, ones, zeros)
        return nb + jnp.sum(mi)

    nb = lax.fori_loop(0, _CHUNKS, count_body, jnp.int32(0))

    def dest_body(k, carry):
        bbc, nbc = carry
        v = at_v[pl.ds(k * 16, 16)]
        m = (v == zeros) | (v == ones) | (v == twos)
        mi = jnp.where(m, ones, zeros)
        cs = jnp.cumsum(mi)
        csn = lanes + ones - cs
        bb_dest = jnp.full((16,), row_base + bbc - 1, jnp.int32) + cs
        nbb_dest = jnp.full((16,), row_base + nb + nbc - 1, jnp.int32) + csn
        dest_v[k] = jnp.where(m, bb_dest, nbb_dest)
        pc = jnp.sum(mi)
        return (bbc + pc, nbc + (16 - pc))

    lax.fori_loop(0, _CHUNKS, dest_body, (jnp.int32(0), jnp.int32(0)))

    src_base = row_base + q * _QROWS
    chunk_base = q * (_QROWS // _SCB)
    rsems = (rsem0, rsem1, rsem2, rsem3, rsem4, rsem5)
    ssems = (ssem0, ssem1, ssem2, ssem3, ssem4, ssem5)

    def read(r):
        s = r % _NSLOT
        return pltpu.async_copy(
            x_hbm.at[pl.ds(src_base + r * _SLOT, _SLOT)],
            ring.at[pl.ds(s * _SLOT, _SLOT)], rsems[s])

    def scatter(r, j):
        s = r % _NSLOT
        c = chunk_base + r * _SPS + j
        return pltpu.async_copy(
            ring.at[pl.ds(s * _SLOT + j * _SCB, _SCB)],
            out_hbm.at[dest_v.at[c]], ssems[s])

    rd = [None] * _NRD
    sd = [[None] * _SPS for _ in range(_NRD)]
    for r in range(5):
        rd[r] = read(r)
    for r in range(_NRD):
        rd[r].wait()
        for j in range(_SPS):
            sd[r][j] = scatter(r, j)
        if r + 5 < _NRD:
            if r >= 1:
                for j in range(_SPS):
                    sd[r - 1][j].wait()
            rd[r + 5] = read(r + 5)
    for r in range(_NRD - 6, _NRD):
        for j in range(_SPS):
            sd[r][j].wait()


def _sc_permute(x2, at32):
    mesh = plsc.VectorSubcoreMesh(core_axis_name="c", subcore_axis_name="s")
    k = pl.kernel(
        _sc_body,
        out_type=jax.ShapeDtypeStruct((_B * _L, _D), jnp.float32),
        mesh=mesh,
        compiler_params=pltpu.CompilerParams(needs_layout_passes=False),
        scratch_types=[
            pltpu.VMEM((_L,), jnp.int32),
            pltpu.VMEM((_CHUNKS, 16), jnp.int32),
            pltpu.VMEM((_NSLOT * _SLOT, _D), jnp.float32),
            pltpu.SemaphoreType.DMA,
            pltpu.SemaphoreType.DMA,
            pltpu.SemaphoreType.DMA,
            pltpu.SemaphoreType.DMA,
            pltpu.SemaphoreType.DMA,
            pltpu.SemaphoreType.DMA,
            pltpu.SemaphoreType.DMA,
            pltpu.SemaphoreType.DMA,
            pltpu.SemaphoreType.DMA,
            pltpu.SemaphoreType.DMA,
            pltpu.SemaphoreType.DMA,
            pltpu.SemaphoreType.DMA,
        ],
    )
    return k(x2, at32)


@jax.jit
def kernel(x, atom_type, aa_type):
    x2 = x.reshape(_B * _L, _D)
    at32 = atom_type.astype(jnp.int32)
    out = _sc_permute(x2, at32)
    return out.reshape(_B, _L, _D)


# speedup vs baseline: 1.0531x; 1.0352x over previous
import jax
import jax.numpy as jnp
from jax import lax
from jax.experimental import pallas as pl
from jax.experimental.pallas import tpu as pltpu, tpu_sc as plsc

_NC, _NS = 2, 16
_NW = _NC * _NS
_B, _L, _D = 8, 2048, 512
_WPB = _NW // _B
_QROWS = _L // _WPB
_CHUNKS = _L // 16
_SLOT = 32
_NSLOT = 6
_NRD = _QROWS // _SLOT
_SCB = 16
_SPS = _SLOT // _SCB


def _sc_body(x_hbm, at_hbm, out_hbm, at_v, dest_v, ring,
             rsem0, rsem1, rsem2, rsem3, rsem4, rsem5,
             ssem0, ssem1, ssem2, ssem3, ssem4, ssem5):
    cid = lax.axis_index("c")
    sid = lax.axis_index("s")
    wid = sid * _NC + cid
    b = wid // _WPB
    q = wid % _WPB

    row_base = b * _L
    src_base = row_base + q * _QROWS
    qchunk = q * (_QROWS // 16)
    rsems = (rsem0, rsem1, rsem2, rsem3, rsem4, rsem5)
    ssems = (ssem0, ssem1, ssem2, ssem3, ssem4, ssem5)

    def read(r):
        s = r % _NSLOT
        return pltpu.async_copy(
            x_hbm.at[pl.ds(src_base + r * _SLOT, _SLOT)],
            ring.at[pl.ds(s * _SLOT, _SLOT)], rsems[s])

    def scatter(r, j):
        s = r % _NSLOT
        c = r * _SPS + j
        return pltpu.async_copy(
            ring.at[pl.ds(s * _SLOT + j * _SCB, _SCB)],
            out_hbm.at[dest_v.at[c]], ssems[s])

    rd = [None] * _NRD
    for r in range(5):
        rd[r] = read(r)

    pltpu.sync_copy(at_hbm.at[b], at_v)

    lanes = jnp.arange(16, dtype=jnp.int32)
    ones = jnp.full((16,), 1, jnp.int32)
    zeros = jnp.full((16,), 0, jnp.int32)
    twos = jnp.full((16,), 2, jnp.int32)

    def count_body(k, carry):
        nbt, bbp = carry
        v = at_v[pl.ds(k * 16, 16)]
        m = (v == zeros) | (v == ones) | (v == twos)
        mi = jnp.where(m, ones, zeros)
        pc = jnp.sum(mi)
        bbp = bbp + jnp.where(k < qchunk, pc, jnp.int32(0))
        return (nbt + pc, bbp)

    nb, bb_pre = lax.fori_loop(0, _CHUNKS, count_body,
                               (jnp.int32(0), jnp.int32(0)))
    nbb_pre = qchunk * 16 - bb_pre

    def dest_body(i, carry):
        bbc, nbc = carry
        v = at_v[pl.ds((qchunk + i) * 16, 16)]
        m = (v == zeros) | (v == ones) | (v == twos)
        mi = jnp.where(m, ones, zeros)
        cs = jnp.cumsum(mi)
        csn = lanes + ones - cs
        bb_dest = jnp.full((16,), row_base + bbc - 1, jnp.int32) + cs
        nbb_dest = jnp.full((16,), row_base + nb + nbc - 1, jnp.int32) + csn
        dest_v[i] = jnp.where(m, bb_dest, nbb_dest)
        pc = jnp.sum(mi)
        return (bbc + pc, nbc + (16 - pc))

    lax.fori_loop(0, _QROWS // 16, dest_body, (bb_pre, nbb_pre))

    sd = [[None] * _SPS for _ in range(_NRD)]
    for r in range(_NRD):
        rd[r].wait()
        for j in range(_SPS):
            sd[r][j] = scatter(r, j)
        if r + 5 < _NRD:
            if r >= 1:
                for j in range(_SPS):
                    sd[r - 1][j].wait()
            rd[r + 5] = read(r + 5)
    for r in range(_NRD - 6, _NRD):
        for j in range(_SPS):
            sd[r][j].wait()


def _sc_permute(x2, at32):
    mesh = plsc.VectorSubcoreMesh(core_axis_name="c", subcore_axis_name="s")
    k = pl.kernel(
        _sc_body,
        out_type=jax.ShapeDtypeStruct((_B * _L, _D), jnp.float32),
        mesh=mesh,
        compiler_params=pltpu.CompilerParams(needs_layout_passes=False),
        scratch_types=[
            pltpu.VMEM((_L,), jnp.int32),
            pltpu.VMEM((_QROWS // 16, 16), jnp.int32),
            pltpu.VMEM((_NSLOT * _SLOT, _D), jnp.float32),
            pltpu.SemaphoreType.DMA,
            pltpu.SemaphoreType.DMA,
            pltpu.SemaphoreType.DMA,
            pltpu.SemaphoreType.DMA,
            pltpu.SemaphoreType.DMA,
            pltpu.SemaphoreType.DMA,
            pltpu.SemaphoreType.DMA,
            pltpu.SemaphoreType.DMA,
            pltpu.SemaphoreType.DMA,
            pltpu.SemaphoreType.DMA,
            pltpu.SemaphoreType.DMA,
            pltpu.SemaphoreType.DMA,
        ],
    )
    return k(x2, at32)


@jax.jit
def kernel(x, atom_type, aa_type):
    x2 = x.reshape(_B * _L, _D)
    at32 = atom_type.astype(jnp.int32)
    out = _sc_permute(x2, at32)
    return out.reshape(_B, _L, _D)
